# baseline (speedup 1.0000x reference)
import functools

import jax
import jax.numpy as jnp
from jax import lax
from jax.experimental import pallas as pl
from jax.experimental.pallas import tpu as pltpu
from jax.experimental.pallas import tpu_sc as plsc

BATCH = 16384
NF = 32
NC, NS = 2, 16
NW = NC * NS
B_PER_W = BATCH // NW
IC = 128
N_IC = B_PER_W // IC


NU = 1000000


def _sc_gather_t(Pl, Ql, Ul, Vl, user_id, item_id):
    mesh = plsc.VectorSubcoreMesh(core_axis_name="c", subcore_axis_name="s")
    out = jax.ShapeDtypeStruct((NF, BATCH), jnp.float32)

    @functools.partial(
        pl.kernel,
        mesh=mesh,
        out_type=(out, out, out, out),
        compiler_params=pltpu.CompilerParams(
            use_tc_tiling_on_sc=False, needs_layout_passes=False),
        scratch_types=[
            pltpu.VMEM((B_PER_W,), jnp.int32),
            pltpu.VMEM((B_PER_W,), jnp.int32),
            pltpu.VMEM((B_PER_W,), jnp.int32),
            pltpu.VMEM((B_PER_W,), jnp.int32),
            pltpu.VMEM((NF, B_PER_W), jnp.float32),
            pltpu.VMEM((NF, B_PER_W), jnp.float32),
            pltpu.VMEM((NF, B_PER_W), jnp.float32),
            pltpu.VMEM((NF, B_PER_W), jnp.float32),
            pltpu.SemaphoreType.DMA,
            pltpu.SemaphoreType.DMA,
            pltpu.SemaphoreType.DMA,
            pltpu.SemaphoreType.DMA,
        ],
    )
    def k(p_hbm, q_hbm, u_hbm, v_hbm, iu_hbm, ii_hbm,
          pmf_hbm, qmf_hbm, pml_hbm, qml_hbm,
          iu_v, ii_v, eu_v, ei_v, pv, qv, uv, vv, sp, sq, su, sv):
        wid = lax.axis_index("s") * NC + lax.axis_index("c")
        base = wid * B_PER_W
        pltpu.sync_copy(iu_hbm.at[pl.ds(base, B_PER_W)], iu_v)
        pltpu.sync_copy(ii_hbm.at[pl.ds(base, B_PER_W)], ii_v)

        @pl.loop(0, NF)
        def _(f):
            off = f * NU

            @pl.loop(0, B_PER_W // 16)
            def _(cc):
                s = pl.ds(cc * 16, 16)
                eu_v[s] = iu_v[s] + off
                ei_v[s] = ii_v[s] + off

            pltpu.async_copy(p_hbm.at[eu_v], pv.at[f], sp)
            pltpu.async_copy(q_hbm.at[ei_v], qv.at[f], sq)
            pltpu.async_copy(u_hbm.at[eu_v], uv.at[f], su)
            pltpu.async_copy(v_hbm.at[ei_v], vv.at[f], sv)
            pltpu.make_async_copy(p_hbm.at[eu_v], pv.at[f], sp).wait()
            pltpu.make_async_copy(q_hbm.at[ei_v], qv.at[f], sq).wait()
            pltpu.make_async_copy(u_hbm.at[eu_v], uv.at[f], su).wait()
            pltpu.make_async_copy(v_hbm.at[ei_v], vv.at[f], sv).wait()

        pltpu.sync_copy(pv, pmf_hbm.at[:, pl.ds(base, B_PER_W)])
        pltpu.sync_copy(qv, qmf_hbm.at[:, pl.ds(base, B_PER_W)])
        pltpu.sync_copy(uv, pml_hbm.at[:, pl.ds(base, B_PER_W)])
        pltpu.sync_copy(vv, qml_hbm.at[:, pl.ds(base, B_PER_W)])

    return k(Pl, Ql, Ul, Vl, user_id, item_id)


def _dotT(w, x):
    return lax.dot_general(w, x, (((0,), (0,)), ((), ())),
                           preferred_element_type=jnp.float32)


def _tc_mlp_body(pmf_ref, qmf_ref, pml_ref, qml_ref,
                 w0_ref, b0_ref, w1_ref, b1_ref, w2_ref, b2_ref,
                 wp_ref, bp_ref, out_ref):
    h = (_dotT(w0_ref[:NF, :], pml_ref[...])
         + _dotT(w0_ref[NF:, :], qml_ref[...])
         + b0_ref[...].T)
    h = jnp.maximum(h, 0.0)
    h = _dotT(w1_ref[...], h) + b1_ref[...].T
    h = jnp.maximum(h, 0.0)
    h = _dotT(w2_ref[...], h) + b2_ref[...].T
    h = jnp.maximum(h, 0.0)
    gmf = pmf_ref[...] * qmf_ref[...]
    out = (_dotT(wp_ref[:NF, :], gmf)
           + _dotT(wp_ref[NF:, :], h)
           + bp_ref[...].T)
    out_ref[...] = out


def _tc_mlp(pmf, qmf, pml, qml, W0, b0, W1, b1, W2, b2, Wp, bp):
    blk = 4096
    grid = (BATCH // blk,)
    in_col = pl.BlockSpec((NF, blk), lambda i: (0, i))
    full = lambda a: pl.BlockSpec(a.shape, lambda i: (0,) * a.ndim)
    return pl.pallas_call(
        _tc_mlp_body,
        grid=grid,
        in_specs=[in_col, in_col, in_col, in_col,
                  full(W0), full(b0), full(W1), full(b1),
                  full(W2), full(b2), full(Wp), full(bp)],
        out_specs=pl.BlockSpec((1, blk), lambda i: (0, i)),
        out_shape=jax.ShapeDtypeStruct((1, BATCH), jnp.float32),
    )(pmf, qmf, pml, qml, W0, b0, W1, b1, W2, b2, Wp, bp)


def kernel(user_id, item_id, P, Q, U, V, W0, b0, W1, b1, W2, b2, Wp, bp):
    pmf, qmf, pml, qml = _sc_gather_t(
        P.T.reshape(-1), Q.T.reshape(-1), U.T.reshape(-1), V.T.reshape(-1),
        user_id, item_id)
    out = _tc_mlp(pmf, qmf, pml, qml,
                  W0, b0.reshape(1, -1), W1, b1.reshape(1, -1),
                  W2, b2.reshape(1, -1), Wp, bp.reshape(1, -1))
    return out.reshape(BATCH)


# speedup vs baseline: 1.0009x; 1.0009x over previous
import functools

import jax
import jax.numpy as jnp
from jax import lax
from jax.experimental import pallas as pl
from jax.experimental.pallas import tpu as pltpu
from jax.experimental.pallas import tpu_sc as plsc

BATCH = 16384
NF = 32
NC, NS = 2, 16
NW = NC * NS
B_PER_W = BATCH // NW
IC = 128
N_IC = B_PER_W // IC


NU = 1000000


def _sc_gather_t(Pl, Ql, Ul, Vl, user_id, item_id):
    mesh = plsc.VectorSubcoreMesh(core_axis_name="c", subcore_axis_name="s")
    out = jax.ShapeDtypeStruct((NF, BATCH), jnp.float32)

    @functools.partial(
        pl.kernel,
        mesh=mesh,
        out_type=(out, out, out, out),
        compiler_params=pltpu.CompilerParams(
            use_tc_tiling_on_sc=False, needs_layout_passes=False),
        scratch_types=[
            pltpu.VMEM((B_PER_W,), jnp.int32),
            pltpu.VMEM((B_PER_W,), jnp.int32),
            pltpu.VMEM((NF, B_PER_W), jnp.float32),
            pltpu.VMEM((NF, B_PER_W), jnp.float32),
            pltpu.VMEM((NF, B_PER_W), jnp.float32),
            pltpu.VMEM((NF, B_PER_W), jnp.float32),
            pltpu.SemaphoreType.DMA,
            pltpu.SemaphoreType.DMA,
            pltpu.SemaphoreType.DMA,
            pltpu.SemaphoreType.DMA,
        ],
    )
    def k(p_hbm, q_hbm, u_hbm, v_hbm, iu_hbm, ii_hbm,
          pmf_hbm, qmf_hbm, pml_hbm, qml_hbm,
          iu_v, ii_v, pv, qv, uv, vv, sp, sq, su, sv):
        wid = lax.axis_index("s") * NC + lax.axis_index("c")
        base = wid * B_PER_W
        pltpu.sync_copy(iu_hbm.at[pl.ds(base, B_PER_W)], iu_v)
        pltpu.sync_copy(ii_hbm.at[pl.ds(base, B_PER_W)], ii_v)

        @pl.loop(0, NF)
        def _(f):
            pltpu.async_copy(p_hbm.at[f].at[iu_v], pv.at[f], sp)
            pltpu.async_copy(q_hbm.at[f].at[ii_v], qv.at[f], sq)
            pltpu.async_copy(u_hbm.at[f].at[iu_v], uv.at[f], su)
            pltpu.async_copy(v_hbm.at[f].at[ii_v], vv.at[f], sv)
            pltpu.make_async_copy(p_hbm.at[f].at[iu_v], pv.at[f], sp).wait()
            pltpu.make_async_copy(q_hbm.at[f].at[ii_v], qv.at[f], sq).wait()
            pltpu.make_async_copy(u_hbm.at[f].at[iu_v], uv.at[f], su).wait()
            pltpu.make_async_copy(v_hbm.at[f].at[ii_v], vv.at[f], sv).wait()

        pltpu.sync_copy(pv, pmf_hbm.at[:, pl.ds(base, B_PER_W)])
        pltpu.sync_copy(qv, qmf_hbm.at[:, pl.ds(base, B_PER_W)])
        pltpu.sync_copy(uv, pml_hbm.at[:, pl.ds(base, B_PER_W)])
        pltpu.sync_copy(vv, qml_hbm.at[:, pl.ds(base, B_PER_W)])

    return k(Pl, Ql, Ul, Vl, user_id, item_id)


def _dotT(w, x):
    return lax.dot_general(w, x, (((0,), (0,)), ((), ())),
                           preferred_element_type=jnp.float32)


def _tc_mlp_body(pmf_ref, qmf_ref, pml_ref, qml_ref,
                 w0_ref, b0_ref, w1_ref, b1_ref, w2_ref, b2_ref,
                 wp_ref, bp_ref, out_ref):
    h = (_dotT(w0_ref[:NF, :], pml_ref[...])
         + _dotT(w0_ref[NF:, :], qml_ref[...])
         + b0_ref[...].T)
    h = jnp.maximum(h, 0.0)
    h = _dotT(w1_ref[...], h) + b1_ref[...].T
    h = jnp.maximum(h, 0.0)
    h = _dotT(w2_ref[...], h) + b2_ref[...].T
    h = jnp.maximum(h, 0.0)
    gmf = pmf_ref[...] * qmf_ref[...]
    out = (_dotT(wp_ref[:NF, :], gmf)
           + _dotT(wp_ref[NF:, :], h)
           + bp_ref[...].T)
    out_ref[...] = out


def _tc_mlp(pmf, qmf, pml, qml, W0, b0, W1, b1, W2, b2, Wp, bp):
    blk = 4096
    grid = (BATCH // blk,)
    in_col = pl.BlockSpec((NF, blk), lambda i: (0, i))
    full = lambda a: pl.BlockSpec(a.shape, lambda i: (0,) * a.ndim)
    return pl.pallas_call(
        _tc_mlp_body,
        grid=grid,
        in_specs=[in_col, in_col, in_col, in_col,
                  full(W0), full(b0), full(W1), full(b1),
                  full(W2), full(b2), full(Wp), full(bp)],
        out_specs=pl.BlockSpec((1, blk), lambda i: (0, i)),
        out_shape=jax.ShapeDtypeStruct((1, BATCH), jnp.float32),
    )(pmf, qmf, pml, qml, W0, b0, W1, b1, W2, b2, Wp, bp)


def kernel(user_id, item_id, P, Q, U, V, W0, b0, W1, b1, W2, b2, Wp, bp):
    pmf, qmf, pml, qml = _sc_gather_t(P.T, Q.T, U.T, V.T, user_id, item_id)
    out = _tc_mlp(pmf, qmf, pml, qml,
                  W0, b0.reshape(1, -1), W1, b1.reshape(1, -1),
                  W2, b2.reshape(1, -1), Wp, bp.reshape(1, -1))
    return out.reshape(BATCH)
